# baseline (speedup 1.0000x reference)
import functools

import jax
import jax.numpy as jnp
from jax import lax
from jax.experimental import pallas as pl
from jax.experimental.pallas import tpu as pltpu
from jax.experimental.pallas import tpu_sc as plsc

_BSZ, _SEQ, _H, _E = 4, 4096, 256, 32
_TOP_K = 1
_ALPHA = 0.001
_SCALE = 1.0
_NROW = _BSZ * _SEQ

_G = 2
_S = 2
_C = _NROW // (_G * _S)
_RSTEP = _C * _S


def _gate_body(*refs):
    x_refs = refs[:_S]
    w_ref = refs[_S]
    idx_ref, wgt_ref, ssum_ref = refs[_S + 1:]
    i = pl.program_id(0)

    @pl.when(i == 0)
    def _init():
        ssum_ref[...] = jnp.zeros_like(ssum_ref)

    w = w_ref[...]
    liota = jax.lax.broadcasted_iota(jnp.int32, (1, 128), 1)
    for s in range(_S):
        x = x_refs[s][...]
        logits = jax.lax.dot_general(
            w, x, (((1,), (1,)), ((), ())), preferred_element_type=jnp.float32
        )
        colmax = jnp.max(logits, axis=0, keepdims=True)
        ex = jnp.exp(logits - colmax)
        denom = jnp.sum(ex, axis=0, keepdims=True)
        eiota = jax.lax.broadcasted_iota(jnp.int32, logits.shape, 0)
        idx = jnp.min(jnp.where(logits == colmax, eiota, _E), axis=0, keepdims=True)
        sl = slice(s * _C, (s + 1) * _C)
        idx_ref[0, 0, sl] = idx[0]
        wgt_ref[0, 0, sl] = (_SCALE / denom)[0]

        probs = ex / denom
        r = i * _S + s
        b = r // (_SEQ // _C)
        bmask = (liota == b).astype(jnp.float32)
        ssum_ref[...] += jnp.sum(probs, axis=1, keepdims=True) * bmask


def _mk_x_spec(s):
    return pl.BlockSpec((_C, _H), lambda i, s=s: (i * _S + s, 0))


def _tc_gate(x, weight):
    return pl.pallas_call(
        _gate_body,
        grid=(_G,),
        in_specs=[_mk_x_spec(s) for s in range(_S)]
        + [pl.BlockSpec((_E, _H), lambda i: (0, 0))],
        out_specs=[
            pl.BlockSpec((1, 1, _RSTEP), lambda i: (i, 0, 0)),
            pl.BlockSpec((1, 1, _RSTEP), lambda i: (i, 0, 0)),
            pl.BlockSpec((_E, 128), lambda i: (0, 0)),
        ],
        out_shape=[
            jax.ShapeDtypeStruct((_G, 1, _RSTEP), jnp.int32),
            jax.ShapeDtypeStruct((_G, 1, _RSTEP), jnp.float32),
            jax.ShapeDtypeStruct((_E, 128), jnp.float32),
        ],
    )(*([x] * _S), weight)


_NSUB = 16
_TPT = _NROW // _NSUB
_VECS = _TPT // 16
_AUX_CONST = _ALPHA * _E / (_SEQ * float(_SEQ) * _BSZ)


def _sc_aux_body(idx_hbm, ssum_hbm, out_hbm, idx_v, hist_v, ssum_v, cnts_v,
                 out_v, stage_hbm):
    c = lax.axis_index("c")
    sid = lax.axis_index("s")
    on_core0 = c == 0

    @pl.when(on_core0)
    def _count():
        pltpu.sync_copy(idx_hbm.at[pl.ds(sid * _TPT, _TPT)], idx_v)
        zero16 = jnp.zeros((16,), jnp.float32)
        hist_v[pl.ds(0, 16)] = zero16
        hist_v[pl.ds(16, 16)] = zero16
        ones = jnp.ones((16,), jnp.float32)
        for k in range(_VECS):
            v = idx_v[pl.ds(k * 16, 16)]
            plsc.addupdate_scatter(hist_v, [v], ones)
        pltpu.sync_copy(hist_v, stage_hbm.at[sid])

    plsc.subcore_barrier()

    @pl.when(jnp.logical_and(on_core0, sid == 0))
    def _finalize():
        pltpu.sync_copy(stage_hbm, cnts_v)
        pltpu.sync_copy(ssum_hbm, ssum_v)
        iota16 = lax.iota(jnp.int32, 16)
        acc = jnp.zeros((16,), jnp.float32)
        tiles_per_batch = _NSUB // _BSZ
        for b in range(_BSZ):
            for h in range(2):
                cnt = jnp.zeros((16,), jnp.float32)
                for t in range(tiles_per_batch):
                    cnt = cnt + cnts_v[b * tiles_per_batch + t, pl.ds(h * 16, 16)]
                rows = iota16 + (h * 16)
                cols = jnp.full((16,), b, jnp.int32)
                sv = plsc.load_gather(ssum_v, [rows, cols])
                acc = acc + cnt * sv
        total = lax.reduce_sum_p.bind(acc, axes=(0,))
        out_v[...] = jnp.where(iota16 == 0,
                               jnp.full((16,), total * _AUX_CONST, jnp.float32),
                               jnp.zeros((16,), jnp.float32))
        pltpu.sync_copy(out_v, out_hbm)


_sc_aux = functools.partial(
    pl.kernel,
    mesh=plsc.VectorSubcoreMesh(core_axis_name="c", subcore_axis_name="s"),
    out_type=jax.ShapeDtypeStruct((16,), jnp.float32),
    compiler_params=pltpu.CompilerParams(needs_layout_passes=False),
    scratch_types=[
        pltpu.VMEM((_TPT,), jnp.int32),
        pltpu.VMEM((_E,), jnp.float32),
        pltpu.VMEM((_E, 128), jnp.float32),
        pltpu.VMEM((_NSUB, _E), jnp.float32),
        pltpu.VMEM((16,), jnp.float32),
        pltpu.HBM((_NSUB, _E), jnp.float32),
    ],
)(_sc_aux_body)


def kernel(hidden_states, weight):
    x = hidden_states.reshape(-1, _H)
    idxs, wgts, ssum = _tc_gate(x, weight)
    idx_flat = idxs.reshape(-1)
    aux = _sc_aux(idx_flat, ssum)
    topk_idx = idxs.reshape(-1, _TOP_K)
    topk_weight = wgts.reshape(-1, _TOP_K)
    return (topk_idx, topk_weight, aux[0])


# speedup vs baseline: 3.2538x; 3.2538x over previous
import jax
import jax.numpy as jnp
from jax.experimental import pallas as pl
from jax.experimental.pallas import tpu as pltpu

_BSZ, _SEQ, _H, _E = 4, 4096, 256, 32
_TOP_K = 1
_ALPHA = 0.001
_SCALE = 1.0
_NROW = _BSZ * _SEQ

_G = 2
_S = 2
_C = _NROW // (_G * _S)
_RSTEP = _C * _S


def _gate_body(*refs):
    x_refs = refs[:_S]
    w_ref = refs[_S]
    idx_ref, wgt_ref, aux_ref, cnt_ref, ssum_ref = refs[_S + 1:]
    i = pl.program_id(0)

    @pl.when(i == 0)
    def _init():
        cnt_ref[...] = jnp.zeros_like(cnt_ref)
        ssum_ref[...] = jnp.zeros_like(ssum_ref)

    w = w_ref[...]
    liota = jax.lax.broadcasted_iota(jnp.int32, (1, 128), 1)
    for s in range(_S):
        x = x_refs[s][...]
        logits = jax.lax.dot_general(
            w, x, (((1,), (1,)), ((), ())), preferred_element_type=jnp.float32
        )
        colmax = jnp.max(logits, axis=0, keepdims=True)
        ex = jnp.exp(logits - colmax)
        denom = jnp.sum(ex, axis=0, keepdims=True)
        eiota = jax.lax.broadcasted_iota(jnp.int32, logits.shape, 0)
        idx = jnp.min(jnp.where(logits == colmax, eiota, _E), axis=0, keepdims=True)
        sl = slice(s * _C, (s + 1) * _C)
        idx_ref[0, 0, sl] = idx[0]
        wgt_ref[0, 0, sl] = (_SCALE / denom)[0]

        probs = ex / denom
        onehot = (eiota == idx).astype(jnp.float32)
        r = i * _S + s
        if _C <= _SEQ:
            b = r // (_SEQ // _C)
            bmask = (liota == b).astype(jnp.float32)
            cnt_ref[...] += jnp.sum(onehot, axis=1, keepdims=True) * bmask
            ssum_ref[...] += jnp.sum(probs, axis=1, keepdims=True) * bmask
        else:
            for j in range(_C // _SEQ):
                b = r * (_C // _SEQ) + j
                seg = slice(j * _SEQ, (j + 1) * _SEQ)
                bmask = (liota == b).astype(jnp.float32)
                cnt_ref[...] += jnp.sum(onehot[:, seg], axis=1, keepdims=True) * bmask
                ssum_ref[...] += jnp.sum(probs[:, seg], axis=1, keepdims=True) * bmask

    @pl.when(i == _G - 1)
    def _fin():
        tot = jnp.sum(cnt_ref[:, : _BSZ] * ssum_ref[:, : _BSZ], keepdims=True)
        aux_ref[...] = tot.reshape(1, 1) * (_ALPHA * _E / (_SEQ * float(_SEQ) * _BSZ))


def _mk_x_spec(s):
    return pl.BlockSpec((_C, _H), lambda i, s=s: (i * _S + s, 0))


def kernel(hidden_states, weight):
    x = hidden_states.reshape(-1, _H)
    idxs, wgts, aux = pl.pallas_call(
        _gate_body,
        grid=(_G,),
        in_specs=[_mk_x_spec(s) for s in range(_S)]
        + [pl.BlockSpec((_E, _H), lambda i: (0, 0))],
        out_specs=[
            pl.BlockSpec((1, 1, _RSTEP), lambda i: (i, 0, 0)),
            pl.BlockSpec((1, 1, _RSTEP), lambda i: (i, 0, 0)),
            pl.BlockSpec((1, 1), lambda i: (0, 0)),
        ],
        out_shape=[
            jax.ShapeDtypeStruct((_G, 1, _RSTEP), jnp.int32),
            jax.ShapeDtypeStruct((_G, 1, _RSTEP), jnp.float32),
            jax.ShapeDtypeStruct((1, 1), jnp.float32),
        ],
        scratch_shapes=[
            pltpu.VMEM((_E, 128), jnp.float32),
            pltpu.VMEM((_E, 128), jnp.float32),
        ],
    )(*([x] * _S), weight)
    topk_idx = idxs.reshape(-1, _TOP_K)
    topk_weight = wgts.reshape(-1, _TOP_K)
    return (topk_idx, topk_weight, aux[0, 0])
